# Initial kernel scaffold; baseline (speedup 1.0000x reference)
import jax
import jax.numpy as jnp
from jax.experimental import pallas as pl


def kernel(tokens, embed, W1, b1, W2, b2):
    raise NotImplementedError("write your pallas kernel here")


# speedup vs baseline: 10.2224x; 10.2224x over previous
import functools

import jax
import jax.numpy as jnp
from jax import lax
from jax.experimental import pallas as pl
from jax.experimental.pallas import tpu as pltpu
from jax.experimental.pallas import tpu_sc as plsc

B = 16384
L = 200
D = 32
FF = 128

NC = 2
NS = 16
NW = NC * NS
BPW = B // NW
LH = L // 2
CH = 64
NCH = BPW // CH


def _pool_sc(tokens3, embed):
    mesh = plsc.VectorSubcoreMesh(core_axis_name="c", subcore_axis_name="s")

    @functools.partial(
        pl.kernel,
        out_type=jax.ShapeDtypeStruct((B, D), jnp.float32),
        mesh=mesh,
        compiler_params=pltpu.CompilerParams(use_tc_tiling_on_sc=False),
        scratch_types=[
            pltpu.VMEM((CH, 2, LH), jnp.int32),
            pltpu.VMEM((2, LH, D), jnp.float32),
            pltpu.VMEM((BPW, D), jnp.float32),
            pltpu.SemaphoreType.DMA,
            pltpu.SemaphoreType.DMA,
        ],
    )
    def pool(tokens_hbm, table_hbm, out_hbm, idx_v, rows_v, pooled_v, sem0, sem1):
        wid = lax.axis_index("s") * NC + lax.axis_index("c")
        base = wid * BPW
        for c in range(NCH):
            pltpu.sync_copy(tokens_hbm.at[pl.ds(base + c * CH, CH)], idx_v)

            @pl.loop(0, CH)
            def _sample(s):
                g0 = pltpu.async_copy(table_hbm.at[idx_v.at[s, 0]], rows_v.at[0], sem0)
                g1 = pltpu.async_copy(table_hbm.at[idx_v.at[s, 1]], rows_v.at[1], sem1)
                g0.wait()
                g1.wait()

                def red(r, accs):
                    a0, a1 = accs
                    a0 = a0 + rows_v[0, r, pl.ds(0, 16)] + rows_v[1, r, pl.ds(0, 16)]
                    a1 = a1 + rows_v[0, r, pl.ds(16, 16)] + rows_v[1, r, pl.ds(16, 16)]
                    return (a0, a1)

                z = jnp.zeros((16,), jnp.float32)
                a0, a1 = lax.fori_loop(0, LH, red, (z, z))
                scale = jnp.float32(1.0 / L)
                pooled_v[s + c * CH, pl.ds(0, 16)] = a0 * scale
                pooled_v[s + c * CH, pl.ds(16, 16)] = a1 * scale

        pltpu.sync_copy(pooled_v, out_hbm.at[pl.ds(base, BPW)])

    return pool(tokens3, embed)


BM = 2048


def _mlp_body(x_ref, w1_ref, b1_ref, w2_ref, b2_ref, o_ref):
    x = x_ref[...]
    h = jnp.maximum(
        jnp.dot(x, w1_ref[...], preferred_element_type=jnp.float32) + b1_ref[...],
        0.0,
    )
    o_ref[...] = jnp.dot(h, w2_ref[...], preferred_element_type=jnp.float32) + b2_ref[...]


def _mlp_tc(x, W1, b1, W2, b2):
    return pl.pallas_call(
        _mlp_body,
        grid=(B // BM,),
        in_specs=[
            pl.BlockSpec((BM, D), lambda i: (i, 0)),
            pl.BlockSpec((D, FF), lambda i: (0, 0)),
            pl.BlockSpec((1, FF), lambda i: (0, 0)),
            pl.BlockSpec((FF, 1), lambda i: (0, 0)),
            pl.BlockSpec((1, 1), lambda i: (0, 0)),
        ],
        out_specs=pl.BlockSpec((BM, 1), lambda i: (i, 0)),
        out_shape=jax.ShapeDtypeStruct((B, 1), jnp.float32),
    )(x, W1, b1.reshape(1, FF), W2, b2.reshape(1, 1))


def kernel(tokens, embed, W1, b1, W2, b2):
    tokens3 = tokens.reshape(B, 2, LH).astype(jnp.int32)
    pooled = _pool_sc(tokens3, embed)
    out = _mlp_tc(pooled, W1, b1, W2, b2)
    return out[:, 0]


# speedup vs baseline: 13.7968x; 1.3497x over previous
import functools

import jax
import jax.numpy as jnp
from jax import lax
from jax.experimental import pallas as pl
from jax.experimental.pallas import tpu as pltpu
from jax.experimental.pallas import tpu_sc as plsc

B = 16384
L = 200
D = 32
FF = 128

NC = 2
NS = 16
NW = NC * NS
BPW = B // NW
LA = 104
LB = L - LA
CH = 128
NCH = BPW // CH


def _pool_sc(tokens1d, embed):
    mesh = plsc.VectorSubcoreMesh(core_axis_name="c", subcore_axis_name="s")

    @functools.partial(
        pl.kernel,
        out_type=jax.ShapeDtypeStruct((B * D,), jnp.float32),
        mesh=mesh,
        compiler_params=pltpu.CompilerParams(use_tc_tiling_on_sc=False),
        scratch_types=[
            pltpu.VMEM((CH * L,), jnp.int32),
            pltpu.VMEM((2, L, D), jnp.float32),
            pltpu.VMEM((BPW * D,), jnp.float32),
            pltpu.SemaphoreType.DMA,
            pltpu.SemaphoreType.DMA,
        ],
    )
    def pool(tokens_hbm, table_hbm, out_hbm, idx_v, rows_v, pooled_v, sem0, sem1):
        wid = lax.axis_index("s") * NC + lax.axis_index("c")
        base = wid * BPW
        sems = (sem0, sem1)

        def issue(s, buf):
            sem = sems[buf]
            a = pltpu.async_copy(
                table_hbm.at[idx_v.at[pl.ds(s * L, LA)]],
                rows_v.at[buf, pl.ds(0, LA)], sem)
            b = pltpu.async_copy(
                table_hbm.at[idx_v.at[pl.ds(s * L + LA, LB)]],
                rows_v.at[buf, pl.ds(LA, LB)], sem)
            return a, b

        def reduce_store(c, s, buf, descs):
            descs[0].wait()
            descs[1].wait()

            def red(r, accs):
                a0, a1 = accs
                a0 = a0 + rows_v[buf, r, pl.ds(0, 16)]
                a1 = a1 + rows_v[buf, r, pl.ds(16, 16)]
                return (a0, a1)

            z = jnp.zeros((16,), jnp.float32)
            a0, a1 = lax.fori_loop(0, L, red, (z, z), unroll=8)
            scale = jnp.float32(1.0 / L)
            o = (s + c * CH) * D
            pooled_v[pl.ds(o, 16)] = a0 * scale
            pooled_v[pl.ds(o + 16, 16)] = a1 * scale

        for c in range(NCH):
            pltpu.sync_copy(
                tokens_hbm.at[pl.ds((base + c * CH) * L, CH * L)], idx_v)
            d0 = issue(0, 0)

            @pl.loop(0, CH, step=2)
            def _pair(k):
                da = issue(k + 1, 1)
                reduce_store(c, k, 0, d0)

                @pl.when(k < CH - 2)
                def _():
                    issue(k + 2, 0)

                reduce_store(c, k + 1, 1, da)

        pltpu.sync_copy(pooled_v, out_hbm.at[pl.ds(base * D, BPW * D)])

    return pool(tokens1d, embed)


BM = 2048


def _mlp_body(x_ref, w1_ref, b1_ref, w2_ref, b2_ref, o_ref):
    x = x_ref[...]
    h = jnp.maximum(
        jnp.dot(x, w1_ref[...], preferred_element_type=jnp.float32) + b1_ref[...],
        0.0,
    )
    o_ref[...] = jnp.dot(h, w2_ref[...], preferred_element_type=jnp.float32) + b2_ref[...]


def _mlp_tc(x, W1, b1, W2, b2):
    return pl.pallas_call(
        _mlp_body,
        grid=(B // BM,),
        in_specs=[
            pl.BlockSpec((BM, D), lambda i: (i, 0)),
            pl.BlockSpec((D, FF), lambda i: (0, 0)),
            pl.BlockSpec((1, FF), lambda i: (0, 0)),
            pl.BlockSpec((FF, 1), lambda i: (0, 0)),
            pl.BlockSpec((1, 1), lambda i: (0, 0)),
        ],
        out_specs=pl.BlockSpec((BM, 1), lambda i: (i, 0)),
        out_shape=jax.ShapeDtypeStruct((B, 1), jnp.float32),
    )(x, W1, b1.reshape(1, FF), W2, b2.reshape(1, 1))


def kernel(tokens, embed, W1, b1, W2, b2):
    tok1 = tokens.astype(jnp.int32).reshape(B * L)
    pooled = _pool_sc(tok1, embed).reshape(B, D)
    out = _mlp_tc(pooled, W1, b1, W2, b2)
    return out[:, 0]


# speedup vs baseline: 13.8094x; 1.0009x over previous
import functools

import jax
import jax.numpy as jnp
from jax import lax
from jax.experimental import pallas as pl
from jax.experimental.pallas import tpu as pltpu
from jax.experimental.pallas import tpu_sc as plsc

B = 16384
L = 200
D = 32
FF = 128

NC = 2
NS = 16
NW = NC * NS
BPW = B // NW
LA = 104
LB = L - LA
CH = 128
NCH = BPW // CH


def _pool_sc(tokens1d, embed):
    mesh = plsc.VectorSubcoreMesh(core_axis_name="c", subcore_axis_name="s")

    @functools.partial(
        pl.kernel,
        out_type=jax.ShapeDtypeStruct((B * D,), jnp.float32),
        mesh=mesh,
        compiler_params=pltpu.CompilerParams(use_tc_tiling_on_sc=False),
        scratch_types=[
            pltpu.VMEM((CH, L), jnp.int32),
            pltpu.VMEM((2, L, D), jnp.float32),
            pltpu.VMEM((BPW * D,), jnp.float32),
            pltpu.SemaphoreType.DMA,
            pltpu.SemaphoreType.DMA,
        ],
    )
    def pool(tokens_hbm, table_hbm, out_hbm, idx_v, rows_v, pooled_v, sem0, sem1):
        wid = lax.axis_index("s") * NC + lax.axis_index("c")
        base = wid * BPW
        sems = (sem0, sem1)

        def issue(s, buf):
            sem = sems[buf]
            a = pltpu.async_copy(
                table_hbm.at[idx_v.at[s, pl.ds(0, LA)]],
                rows_v.at[buf, pl.ds(0, LA)], sem)
            b = pltpu.async_copy(
                table_hbm.at[idx_v.at[s, pl.ds(LA, LB)]],
                rows_v.at[buf, pl.ds(LA, LB)], sem)
            return a, b

        def reduce_store(c, s, buf, descs):
            descs[0].wait()
            descs[1].wait()

            def red(r, accs):
                a0, a1 = accs
                a0 = a0 + rows_v[buf, r, pl.ds(0, 16)]
                a1 = a1 + rows_v[buf, r, pl.ds(16, 16)]
                return (a0, a1)

            z = jnp.zeros((16,), jnp.float32)
            a0, a1 = lax.fori_loop(0, L, red, (z, z), unroll=8)
            scale = jnp.float32(1.0 / L)
            o = (s + c * CH) * D
            pooled_v[pl.ds(o, 16)] = a0 * scale
            pooled_v[pl.ds(o + 16, 16)] = a1 * scale

        for c in range(NCH):
            pltpu.sync_copy(
                tokens_hbm.at[pl.ds(base + c * CH, CH)], idx_v)
            d0 = issue(0, 0)

            @pl.loop(0, CH, step=2)
            def _pair(k):
                da = issue(k + 1, 1)
                reduce_store(c, k, 0, d0)

                @pl.when(k < CH - 2)
                def _():
                    issue(k + 2, 0)

                reduce_store(c, k + 1, 1, da)

        pltpu.sync_copy(pooled_v, out_hbm.at[pl.ds(base * D, BPW * D)])

    return pool(tokens1d, embed)


BM = 2048


def _mlp_body(x_ref, w1_ref, b1_ref, w2_ref, b2_ref, o_ref):
    x = x_ref[...]
    h = jnp.maximum(
        jnp.dot(x, w1_ref[...], preferred_element_type=jnp.float32) + b1_ref[...],
        0.0,
    )
    o_ref[...] = jnp.dot(h, w2_ref[...], preferred_element_type=jnp.float32) + b2_ref[...]


def _mlp_tc(x, W1, b1, W2, b2):
    return pl.pallas_call(
        _mlp_body,
        grid=(B // BM,),
        in_specs=[
            pl.BlockSpec((BM, D), lambda i: (i, 0)),
            pl.BlockSpec((D, FF), lambda i: (0, 0)),
            pl.BlockSpec((1, FF), lambda i: (0, 0)),
            pl.BlockSpec((FF, 1), lambda i: (0, 0)),
            pl.BlockSpec((1, 1), lambda i: (0, 0)),
        ],
        out_specs=pl.BlockSpec((BM, 1), lambda i: (i, 0)),
        out_shape=jax.ShapeDtypeStruct((B, 1), jnp.float32),
    )(x, W1, b1.reshape(1, FF), W2, b2.reshape(1, 1))


def kernel(tokens, embed, W1, b1, W2, b2):
    pooled = _pool_sc(tokens.astype(jnp.int32), embed).reshape(B, D)
    out = _mlp_tc(pooled, W1, b1, W2, b2)
    return out[:, 0]
